# Initial kernel scaffold; baseline (speedup 1.0000x reference)
import jax
import jax.numpy as jnp
from jax.experimental import pallas as pl


def kernel(x, edge_index, W1, b1, W2, b2, W_out, b_out):
    raise NotImplementedError("write your pallas kernel here")


# speedup vs baseline: 7.5551x; 7.5551x over previous
import functools

import jax
import jax.numpy as jnp
from jax import lax
from jax.experimental import pallas as pl
from jax.experimental.pallas import tpu as pltpu
from jax.experimental.pallas import tpu_sc as plsc

N = 10000
E = 320000
D = 128
C = 64
NC = 2
NS = 16
CHUNK = 128
NCHUNK = 80
E_PAD = NC * NS * NCHUNK * CHUNK
N_PAD = 10240
ROWS_PER_TILE = N_PAD // NS
DEG_W = 128

_sc_mesh = plsc.VectorSubcoreMesh(core_axis_name="c", subcore_axis_name="s")


@functools.partial(
    pl.kernel,
    mesh=_sc_mesh,
    out_type=jax.ShapeDtypeStruct((NC, N_PAD, DEG_W), jnp.float32),
    scratch_types=[
        pltpu.VMEM((NCHUNK, CHUNK), jnp.int32),
        pltpu.VMEM((CHUNK, DEG_W), jnp.float32),
        pltpu.VMEM_SHARED((N_PAD, DEG_W), jnp.float32),
    ],
)
def _sc_degree(col_hbm, ones_hbm, zeros_hbm, out_hbm, col_v, ones_v, acc_sh):
    c = lax.axis_index("c")
    s = lax.axis_index("s")
    pltpu.sync_copy(col_hbm.at[c, s], col_v)
    pltpu.sync_copy(ones_hbm, ones_v)
    pltpu.sync_copy(zeros_hbm, acc_sh.at[pl.ds(s * ROWS_PER_TILE, ROWS_PER_TILE)])
    plsc.subcore_barrier()

    def body(j, carry):
        pltpu.sync_copy(ones_v, acc_sh.at[col_v.at[j]], add=True)
        return carry

    lax.fori_loop(0, NCHUNK, body, 0)
    plsc.subcore_barrier()
    pltpu.sync_copy(
        acc_sh.at[pl.ds(s * ROWS_PER_TILE, ROWS_PER_TILE)],
        out_hbm.at[c].at[pl.ds(s * ROWS_PER_TILE, ROWS_PER_TILE)],
    )


@functools.partial(
    pl.kernel,
    mesh=_sc_mesh,
    out_type=jax.ShapeDtypeStruct((NC, N_PAD, D), jnp.float32),
    scratch_types=[
        pltpu.VMEM((NCHUNK, CHUNK), jnp.int32),
        pltpu.VMEM((NCHUNK, CHUNK), jnp.int32),
        pltpu.VMEM((CHUNK, D), jnp.float32),
        pltpu.VMEM_SHARED((N_PAD, D), jnp.float32),
        pltpu.SemaphoreType.DMA,
    ],
)
def _sc_propagate(y_hbm, row_hbm, col_hbm, zeros_hbm, out_hbm,
                  row_v, col_v, msg_v, acc_sh, sem):
    c = lax.axis_index("c")
    s = lax.axis_index("s")
    pltpu.sync_copy(row_hbm.at[c, s], row_v)
    pltpu.sync_copy(col_hbm.at[c, s], col_v)
    pltpu.sync_copy(zeros_hbm, acc_sh.at[pl.ds(s * ROWS_PER_TILE, ROWS_PER_TILE)])
    plsc.subcore_barrier()

    def body(j, carry):
        pltpu.async_copy(y_hbm.at[row_v.at[j]], msg_v, sem).wait()
        pltpu.sync_copy(msg_v, acc_sh.at[col_v.at[j]], add=True)
        return carry

    lax.fori_loop(0, NCHUNK, body, 0)
    plsc.subcore_barrier()
    pltpu.sync_copy(
        acc_sh.at[pl.ds(s * ROWS_PER_TILE, ROWS_PER_TILE)],
        out_hbm.at[c].at[pl.ds(s * ROWS_PER_TILE, ROWS_PER_TILE)],
    )


RB = 1000


def _dis_block(dp_ref):
    d = dp_ref[0, :, 0:1] + dp_ref[1, :, 0:1] + 1.0
    return lax.rsqrt(d)


def _tc1_body(x_ref, w_ref, dp_ref, y_ref):
    dis = _dis_block(dp_ref)
    y_ref[...] = jnp.dot(x_ref[...], w_ref[...],
                         preferred_element_type=jnp.float32) * dis


def _tc2_body(a_ref, y_ref, dp_ref, b_ref, w_ref, o_ref):
    dis = _dis_block(dp_ref)
    h = jnp.maximum(dis * (a_ref[0] + a_ref[1] + y_ref[...]) + b_ref[...], 0.0)
    o_ref[...] = jnp.dot(h, w_ref[...], preferred_element_type=jnp.float32) * dis


def _tc3_body(a_ref, y_ref, dp_ref, b_ref, w_ref, bo_ref, o_ref):
    dis = _dis_block(dp_ref)
    h = jnp.maximum(dis * (a_ref[0] + a_ref[1] + y_ref[...]) + b_ref[...], 0.0)
    o_ref[...] = jnp.dot(h, w_ref[...],
                         preferred_element_type=jnp.float32) + bo_ref[...]


_dp_spec = pl.BlockSpec((NC, RB, DEG_W), lambda i: (0, i, 0))
_acc_spec = pl.BlockSpec((NC, RB, D), lambda i: (0, i, 0))
_rowblk = pl.BlockSpec((RB, D), lambda i: (i, 0))

_tc1 = pl.pallas_call(
    _tc1_body,
    grid=(N // RB,),
    in_specs=[
        _rowblk,
        pl.BlockSpec((D, D), lambda i: (0, 0)),
        _dp_spec,
    ],
    out_specs=_rowblk,
    out_shape=jax.ShapeDtypeStruct((N, D), jnp.float32),
)

_tc2 = pl.pallas_call(
    _tc2_body,
    grid=(N // RB,),
    in_specs=[
        _acc_spec,
        _rowblk,
        _dp_spec,
        pl.BlockSpec((1, D), lambda i: (0, 0)),
        pl.BlockSpec((D, D), lambda i: (0, 0)),
    ],
    out_specs=_rowblk,
    out_shape=jax.ShapeDtypeStruct((N, D), jnp.float32),
)

_tc3 = pl.pallas_call(
    _tc3_body,
    grid=(N // RB,),
    in_specs=[
        _acc_spec,
        _rowblk,
        _dp_spec,
        pl.BlockSpec((1, D), lambda i: (0, 0)),
        pl.BlockSpec((D, C), lambda i: (0, 0)),
        pl.BlockSpec((1, C), lambda i: (0, 0)),
    ],
    out_specs=pl.BlockSpec((RB, C), lambda i: (i, 0)),
    out_shape=jax.ShapeDtypeStruct((N, C), jnp.float32),
)


def kernel(x, edge_index, W1, b1, W2, b2, W_out, b_out):
    row = edge_index[0].astype(jnp.int32)
    col = edge_index[1].astype(jnp.int32)
    row_p = jnp.concatenate(
        [row, jnp.zeros((E_PAD - E,), jnp.int32)]).reshape(NC, NS, NCHUNK, CHUNK)
    col_p = jnp.concatenate(
        [col, jnp.full((E_PAD - E,), N, jnp.int32)]).reshape(NC, NS, NCHUNK, CHUNK)

    onesW = jnp.ones((CHUNK, DEG_W), jnp.float32)
    zerosD = jnp.zeros((ROWS_PER_TILE, D), jnp.float32)

    deg_parts = _sc_degree(col_p, onesW, zerosD)
    y1 = _tc1(x, W1, deg_parts)
    acc1 = _sc_propagate(y1, row_p, col_p, zerosD)
    y2 = _tc2(acc1, y1, deg_parts, b1.reshape(1, D), W2)
    acc2 = _sc_propagate(y2, row_p, col_p, zerosD)
    out = _tc3(acc2, y2, deg_parts, b2.reshape(1, D), W_out.T, b_out.reshape(1, C))
    return out


# speedup vs baseline: 9.3180x; 1.2333x over previous
import functools

import jax
import jax.numpy as jnp
from jax import lax
from jax.experimental import pallas as pl
from jax.experimental.pallas import tpu as pltpu
from jax.experimental.pallas import tpu_sc as plsc

N = 10000
E = 320000
D = 128
C = 64
NC = 2
NS = 16
CHUNK = 128
NCHUNK = 80
E_PAD = NC * NS * NCHUNK * CHUNK
N_PAD = 10240
ROWS_PER_TILE = N_PAD // NS
DEG_W = 128

_sc_mesh = plsc.VectorSubcoreMesh(core_axis_name="c", subcore_axis_name="s")


@functools.partial(
    pl.kernel,
    mesh=_sc_mesh,
    out_type=jax.ShapeDtypeStruct((NC, N_PAD, DEG_W), jnp.float32),
    scratch_types=[
        pltpu.VMEM((NCHUNK, CHUNK), jnp.int32),
        pltpu.VMEM((CHUNK, DEG_W), jnp.float32),
        pltpu.VMEM_SHARED((N_PAD, DEG_W), jnp.float32),
    ],
)
def _sc_degree(col_hbm, ones_hbm, zeros_hbm, out_hbm, col_v, ones_v, acc_sh):
    c = lax.axis_index("c")
    s = lax.axis_index("s")
    pltpu.sync_copy(col_hbm.at[c, s], col_v)
    pltpu.sync_copy(ones_hbm, ones_v)
    pltpu.sync_copy(zeros_hbm, acc_sh.at[pl.ds(s * ROWS_PER_TILE, ROWS_PER_TILE)])
    plsc.subcore_barrier()

    def body(j, carry):
        pltpu.sync_copy(ones_v, acc_sh.at[col_v.at[j]], add=True)
        return carry

    lax.fori_loop(0, NCHUNK, body, 0)
    plsc.subcore_barrier()
    pltpu.sync_copy(
        acc_sh.at[pl.ds(s * ROWS_PER_TILE, ROWS_PER_TILE)],
        out_hbm.at[c].at[pl.ds(s * ROWS_PER_TILE, ROWS_PER_TILE)],
    )


KGRP = 1


@functools.partial(
    pl.kernel,
    mesh=_sc_mesh,
    out_type=jax.ShapeDtypeStruct((NC, N_PAD, D), jnp.float32),
    scratch_types=[
        pltpu.VMEM((NCHUNK, CHUNK), jnp.int32),
        pltpu.VMEM((NCHUNK, CHUNK), jnp.int32),
        pltpu.VMEM((KGRP, CHUNK, D), jnp.float32),
        pltpu.VMEM_SHARED((N_PAD, D), jnp.float32),
        pltpu.SemaphoreType.DMA,
    ],
)
def _sc_propagate(y_hbm, row_hbm, col_hbm, zeros_hbm, out_hbm,
                  row_v, col_v, msg_v, acc_sh, sem):
    c = lax.axis_index("c")
    s = lax.axis_index("s")
    pltpu.sync_copy(row_hbm.at[c, s], row_v)
    pltpu.sync_copy(col_hbm.at[c, s], col_v)
    pltpu.sync_copy(zeros_hbm, acc_sh.at[pl.ds(s * ROWS_PER_TILE, ROWS_PER_TILE)])
    plsc.subcore_barrier()

    def body(g, carry):
        cps = [pltpu.async_copy(y_hbm.at[row_v.at[g * KGRP + k]],
                                msg_v.at[k], sem)
               for k in range(KGRP)]
        for cp in cps:
            cp.wait()
        for k in range(KGRP):
            pltpu.sync_copy(msg_v.at[k], acc_sh.at[col_v.at[g * KGRP + k]],
                            add=True)
        return carry

    lax.fori_loop(0, NCHUNK // KGRP, body, 0)
    plsc.subcore_barrier()
    pltpu.sync_copy(
        acc_sh.at[pl.ds(s * ROWS_PER_TILE, ROWS_PER_TILE)],
        out_hbm.at[c].at[pl.ds(s * ROWS_PER_TILE, ROWS_PER_TILE)],
    )


RB = 1000


def _dis_block(dp_ref):
    d = dp_ref[0, :, 0:1] + dp_ref[1, :, 0:1] + 1.0
    return lax.rsqrt(d)


def _tc1_body(x_ref, w_ref, dp_ref, y_ref):
    dis = _dis_block(dp_ref)
    y_ref[...] = jnp.dot(x_ref[...], w_ref[...],
                         preferred_element_type=jnp.float32) * dis


def _tc2_body(a_ref, y_ref, dp_ref, b_ref, w_ref, o_ref):
    dis = _dis_block(dp_ref)
    h = jnp.maximum(dis * (a_ref[0] + a_ref[1] + y_ref[...]) + b_ref[...], 0.0)
    o_ref[...] = jnp.dot(h, w_ref[...], preferred_element_type=jnp.float32) * dis


def _tc3_body(a_ref, y_ref, dp_ref, b_ref, w_ref, bo_ref, o_ref):
    dis = _dis_block(dp_ref)
    h = jnp.maximum(dis * (a_ref[0] + a_ref[1] + y_ref[...]) + b_ref[...], 0.0)
    o_ref[...] = jnp.dot(h, w_ref[...],
                         preferred_element_type=jnp.float32) + bo_ref[...]


_dp_spec = pl.BlockSpec((NC, RB, DEG_W), lambda i: (0, i, 0))
_acc_spec = pl.BlockSpec((NC, RB, D), lambda i: (0, i, 0))
_rowblk = pl.BlockSpec((RB, D), lambda i: (i, 0))

_tc1 = pl.pallas_call(
    _tc1_body,
    grid=(N // RB,),
    in_specs=[
        _rowblk,
        pl.BlockSpec((D, D), lambda i: (0, 0)),
        _dp_spec,
    ],
    out_specs=_rowblk,
    out_shape=jax.ShapeDtypeStruct((N, D), jnp.float32),
)

_tc2 = pl.pallas_call(
    _tc2_body,
    grid=(N // RB,),
    in_specs=[
        _acc_spec,
        _rowblk,
        _dp_spec,
        pl.BlockSpec((1, D), lambda i: (0, 0)),
        pl.BlockSpec((D, D), lambda i: (0, 0)),
    ],
    out_specs=_rowblk,
    out_shape=jax.ShapeDtypeStruct((N, D), jnp.float32),
)

_tc3 = pl.pallas_call(
    _tc3_body,
    grid=(N // RB,),
    in_specs=[
        _acc_spec,
        _rowblk,
        _dp_spec,
        pl.BlockSpec((1, D), lambda i: (0, 0)),
        pl.BlockSpec((D, C), lambda i: (0, 0)),
        pl.BlockSpec((1, C), lambda i: (0, 0)),
    ],
    out_specs=pl.BlockSpec((RB, C), lambda i: (i, 0)),
    out_shape=jax.ShapeDtypeStruct((N, C), jnp.float32),
)


def kernel(x, edge_index, W1, b1, W2, b2, W_out, b_out):
    row = edge_index[0].astype(jnp.int32)
    col = edge_index[1].astype(jnp.int32)
    n_tiles = NC * NS
    pad_per_tile = E_PAD // n_tiles - E // n_tiles
    pad_cols = jnp.broadcast_to(N + jnp.arange(pad_per_tile, dtype=jnp.int32),
                                (n_tiles, pad_per_tile))
    row_p = jnp.concatenate(
        [row.reshape(n_tiles, E // n_tiles),
         jnp.zeros((n_tiles, pad_per_tile), jnp.int32)],
        axis=1).reshape(NC, NS, NCHUNK, CHUNK)
    col_p = jnp.concatenate(
        [col.reshape(n_tiles, E // n_tiles), pad_cols],
        axis=1).reshape(NC, NS, NCHUNK, CHUNK)

    onesW = jnp.ones((CHUNK, DEG_W), jnp.float32)
    zerosD = jnp.zeros((ROWS_PER_TILE, D), jnp.float32)

    deg_parts = _sc_degree(col_p, onesW, zerosD)
    y1 = _tc1(x, W1, deg_parts)
    acc1 = _sc_propagate(y1, row_p, col_p, zerosD)
    y2 = _tc2(acc1, y1, deg_parts, b1.reshape(1, D), W2)
    acc2 = _sc_propagate(y2, row_p, col_p, zerosD)
    out = _tc3(acc2, y2, deg_parts, b2.reshape(1, D), W_out.T, b_out.reshape(1, C))
    return out


# speedup vs baseline: 19.5941x; 2.1028x over previous
import functools

import jax
import jax.numpy as jnp
from jax import lax
from jax.experimental import pallas as pl
from jax.experimental.pallas import tpu as pltpu
from jax.experimental.pallas import tpu_sc as plsc

N = 10000
E = 320000
D = 128
C = 64
NC = 2
NS = 16
CHUNK = 128
NCHUNK = 80
E_PAD = NC * NS * NCHUNK * CHUNK
N_PAD = 10240
ROWS_PER_TILE = N_PAD // NS
DEG_W = 128

_sc_mesh = plsc.VectorSubcoreMesh(core_axis_name="c", subcore_axis_name="s")


@functools.partial(
    pl.kernel,
    mesh=_sc_mesh,
    out_type=jax.ShapeDtypeStruct((NC, N_PAD, DEG_W), jnp.float32),
    scratch_types=[
        pltpu.VMEM((NCHUNK, CHUNK), jnp.int32),
        pltpu.VMEM((CHUNK, DEG_W), jnp.float32),
        pltpu.VMEM_SHARED((N_PAD, DEG_W), jnp.float32),
    ],
)
def _sc_degree(col_hbm, ones_hbm, zeros_hbm, out_hbm, col_v, ones_v, acc_sh):
    c = lax.axis_index("c")
    s = lax.axis_index("s")
    pltpu.sync_copy(col_hbm.at[c, s], col_v)
    pltpu.sync_copy(ones_hbm, ones_v)
    pltpu.sync_copy(zeros_hbm, acc_sh.at[pl.ds(s * ROWS_PER_TILE, ROWS_PER_TILE)])
    plsc.subcore_barrier()

    def body(j, carry):
        pltpu.sync_copy(ones_v, acc_sh.at[col_v.at[j]], add=True)
        return carry

    lax.fori_loop(0, NCHUNK, body, 0)
    plsc.subcore_barrier()
    pltpu.sync_copy(
        acc_sh.at[pl.ds(s * ROWS_PER_TILE, ROWS_PER_TILE)],
        out_hbm.at[c].at[pl.ds(s * ROWS_PER_TILE, ROWS_PER_TILE)],
    )


KGRP = 1


@functools.partial(
    pl.kernel,
    mesh=_sc_mesh,
    out_type=jax.ShapeDtypeStruct((NC, N_PAD, D), jnp.float32),
    scratch_types=[
        pltpu.VMEM((NCHUNK, CHUNK), jnp.int32),
        pltpu.VMEM((NCHUNK, CHUNK), jnp.int32),
        pltpu.VMEM((KGRP, CHUNK, D), jnp.float32),
        pltpu.VMEM_SHARED((N_PAD, D), jnp.float32),
        pltpu.SemaphoreType.DMA,
    ],
)
def _sc_propagate(y_hbm, row_hbm, col_hbm, zeros_hbm, out_hbm,
                  row_v, col_v, msg_v, acc_sh, sem):
    c = lax.axis_index("c")
    s = lax.axis_index("s")
    pltpu.sync_copy(row_hbm.at[c, s], row_v)
    pltpu.sync_copy(col_hbm.at[c, s], col_v)
    pltpu.sync_copy(zeros_hbm, acc_sh.at[pl.ds(s * ROWS_PER_TILE, ROWS_PER_TILE)])
    plsc.subcore_barrier()

    def body(g, carry):
        cps = [pltpu.async_copy(y_hbm.at[row_v.at[g * KGRP + k]],
                                msg_v.at[k], sem)
               for k in range(KGRP)]
        for cp in cps:
            cp.wait()
        for k in range(KGRP):
            pltpu.sync_copy(msg_v.at[k], acc_sh.at[col_v.at[g * KGRP + k]],
                            add=True)
        return carry

    lax.fori_loop(0, NCHUNK // KGRP, body, 0)
    plsc.subcore_barrier()
    pltpu.sync_copy(
        acc_sh.at[pl.ds(s * ROWS_PER_TILE, ROWS_PER_TILE)],
        out_hbm.at[c].at[pl.ds(s * ROWS_PER_TILE, ROWS_PER_TILE)],
    )


RB = 1000


def _dis_block(dp_ref):
    d = dp_ref[0, :, 0:1] + dp_ref[1, :, 0:1] + 1.0
    return lax.rsqrt(d)


def _tc1_body(x_ref, w_ref, dp_ref, y_ref):
    dis = _dis_block(dp_ref)
    y_ref[...] = jnp.dot(x_ref[...], w_ref[...],
                         preferred_element_type=jnp.float32) * dis


def _tc2_body(a_ref, y_ref, dp_ref, b_ref, w_ref, o_ref):
    dis = _dis_block(dp_ref)
    h = jnp.maximum(dis * (a_ref[0] + a_ref[1] + y_ref[...]) + b_ref[...], 0.0)
    o_ref[...] = jnp.dot(h, w_ref[...], preferred_element_type=jnp.float32) * dis


def _tc3_body(a_ref, y_ref, dp_ref, b_ref, w_ref, bo_ref, o_ref):
    dis = _dis_block(dp_ref)
    h = jnp.maximum(dis * (a_ref[0] + a_ref[1] + y_ref[...]) + b_ref[...], 0.0)
    o_ref[...] = jnp.dot(h, w_ref[...],
                         preferred_element_type=jnp.float32) + bo_ref[...]


_dp_spec = pl.BlockSpec((NC, RB, DEG_W), lambda i: (0, i, 0))
_acc_spec = pl.BlockSpec((NC, RB, D), lambda i: (0, i, 0))
_rowblk = pl.BlockSpec((RB, D), lambda i: (i, 0))

_tc1 = pl.pallas_call(
    _tc1_body,
    grid=(N // RB,),
    in_specs=[
        _rowblk,
        pl.BlockSpec((D, D), lambda i: (0, 0)),
        _dp_spec,
    ],
    out_specs=_rowblk,
    out_shape=jax.ShapeDtypeStruct((N, D), jnp.float32),
)

_tc2 = pl.pallas_call(
    _tc2_body,
    grid=(N // RB,),
    in_specs=[
        _acc_spec,
        _rowblk,
        _dp_spec,
        pl.BlockSpec((1, D), lambda i: (0, 0)),
        pl.BlockSpec((D, D), lambda i: (0, 0)),
    ],
    out_specs=_rowblk,
    out_shape=jax.ShapeDtypeStruct((N, D), jnp.float32),
)

_tc3 = pl.pallas_call(
    _tc3_body,
    grid=(N // RB,),
    in_specs=[
        _acc_spec,
        _rowblk,
        _dp_spec,
        pl.BlockSpec((1, D), lambda i: (0, 0)),
        pl.BlockSpec((D, C), lambda i: (0, 0)),
        pl.BlockSpec((1, C), lambda i: (0, 0)),
    ],
    out_specs=pl.BlockSpec((RB, C), lambda i: (i, 0)),
    out_shape=jax.ShapeDtypeStruct((N, C), jnp.float32),
)


def kernel(x, edge_index, W1, b1, W2, b2, W_out, b_out):
    row = edge_index[0].astype(jnp.int32)
    col = edge_index[1].astype(jnp.int32)
    n_tiles = NC * NS
    pad_per_tile = E_PAD // n_tiles - E // n_tiles
    pad_cols = jnp.broadcast_to(N + jnp.arange(pad_per_tile, dtype=jnp.int32),
                                (n_tiles, pad_per_tile))
    pad_rows = (jnp.arange(n_tiles, dtype=jnp.int32)[:, None] * pad_per_tile
                + jnp.arange(pad_per_tile, dtype=jnp.int32)[None, :]) % N
    row_p = jnp.concatenate(
        [row.reshape(n_tiles, E // n_tiles), pad_rows],
        axis=1).reshape(NC, NS, NCHUNK, CHUNK)
    col_p = jnp.concatenate(
        [col.reshape(n_tiles, E // n_tiles), pad_cols],
        axis=1).reshape(NC, NS, NCHUNK, CHUNK)

    onesW = jnp.ones((CHUNK, DEG_W), jnp.float32)
    zerosD = jnp.zeros((ROWS_PER_TILE, D), jnp.float32)

    deg_parts = _sc_degree(col_p, onesW, zerosD)
    y1 = _tc1(x, W1, deg_parts)
    acc1 = _sc_propagate(y1, row_p, col_p, zerosD)
    y2 = _tc2(acc1, y1, deg_parts, b1.reshape(1, D), W2)
    acc2 = _sc_propagate(y2, row_p, col_p, zerosD)
    out = _tc3(acc2, y2, deg_parts, b2.reshape(1, D), W_out.T, b_out.reshape(1, C))
    return out


# speedup vs baseline: 26.4515x; 1.3500x over previous
import functools

import jax
import jax.numpy as jnp
from jax import lax
from jax.experimental import pallas as pl
from jax.experimental.pallas import tpu as pltpu
from jax.experimental.pallas import tpu_sc as plsc

N = 10000
E = 320000
D = 128
C = 64
NC = 2
NS = 16
CHUNK = 128
NCHUNK = 80
E_PAD = NC * NS * NCHUNK * CHUNK
N_PAD = 10240
ROWS_PER_TILE = N_PAD // NS
DEG_W = 128

_sc_mesh = plsc.VectorSubcoreMesh(core_axis_name="c", subcore_axis_name="s")


@functools.partial(
    pl.kernel,
    mesh=_sc_mesh,
    out_type=jax.ShapeDtypeStruct((NC, N_PAD, DEG_W), jnp.float32),
    scratch_types=[
        pltpu.VMEM((NCHUNK, CHUNK), jnp.int32),
        pltpu.VMEM((CHUNK, DEG_W), jnp.float32),
        pltpu.VMEM_SHARED((N_PAD, DEG_W), jnp.float32),
    ],
)
def _sc_degree(col_hbm, ones_hbm, zeros_hbm, out_hbm, col_v, ones_v, acc_sh):
    c = lax.axis_index("c")
    s = lax.axis_index("s")
    pltpu.sync_copy(col_hbm.at[c, s], col_v)
    pltpu.sync_copy(ones_hbm, ones_v)
    pltpu.sync_copy(zeros_hbm, acc_sh.at[pl.ds(s * ROWS_PER_TILE, ROWS_PER_TILE)])
    plsc.subcore_barrier()

    def body(j, carry):
        pltpu.sync_copy(ones_v, acc_sh.at[col_v.at[j]], add=True)
        return carry

    lax.fori_loop(0, NCHUNK, body, 0)
    plsc.subcore_barrier()
    pltpu.sync_copy(
        acc_sh.at[pl.ds(s * ROWS_PER_TILE, ROWS_PER_TILE)],
        out_hbm.at[c].at[pl.ds(s * ROWS_PER_TILE, ROWS_PER_TILE)],
    )


NPHASE = 2
PCHUNK = NCHUNK // NPHASE


@functools.partial(
    pl.kernel,
    mesh=_sc_mesh,
    out_type=jax.ShapeDtypeStruct((NC, N_PAD, D), jnp.float32),
    scratch_types=[
        pltpu.VMEM((PCHUNK, CHUNK), jnp.int32),
        pltpu.VMEM((PCHUNK, CHUNK), jnp.int32),
        pltpu.VMEM((CHUNK, D), jnp.float32),
        pltpu.VMEM((CHUNK, D), jnp.float32),
        pltpu.VMEM_SHARED((N_PAD, D), jnp.float32),
        pltpu.SemaphoreType.DMA,
        pltpu.SemaphoreType.DMA,
    ],
)
def _sc_propagate(y_hbm, row_hbm, col_hbm, zeros_hbm, out_hbm,
                  row_v, col_v, msg0_v, msg1_v, acc_sh, sem0, sem1):
    c = lax.axis_index("c")
    s = lax.axis_index("s")
    pltpu.sync_copy(zeros_hbm, acc_sh.at[pl.ds(s * ROWS_PER_TILE, ROWS_PER_TILE)])
    plsc.subcore_barrier()

    for p in range(NPHASE):
        pltpu.sync_copy(row_hbm.at[c, s].at[pl.ds(p * PCHUNK, PCHUNK)], row_v)
        pltpu.sync_copy(col_hbm.at[c, s].at[pl.ds(p * PCHUNK, PCHUNK)], col_v)
        pltpu.async_copy(y_hbm.at[row_v.at[0]], msg0_v, sem0)

        def body(g, carry):
            j0 = 2 * g
            pltpu.async_copy(y_hbm.at[row_v.at[j0 + 1]], msg1_v, sem1)
            pltpu.make_async_copy(y_hbm.at[row_v.at[j0]], msg0_v, sem0).wait()
            pltpu.sync_copy(msg0_v, acc_sh.at[col_v.at[j0]], add=True)

            @pl.when(g < PCHUNK // 2 - 1)
            def _():
                pltpu.async_copy(y_hbm.at[row_v.at[j0 + 2]], msg0_v, sem0)

            pltpu.make_async_copy(y_hbm.at[row_v.at[j0 + 1]], msg1_v, sem1).wait()
            pltpu.sync_copy(msg1_v, acc_sh.at[col_v.at[j0 + 1]], add=True)
            return carry

        lax.fori_loop(0, PCHUNK // 2, body, 0)
    plsc.subcore_barrier()
    pltpu.sync_copy(
        acc_sh.at[pl.ds(s * ROWS_PER_TILE, ROWS_PER_TILE)],
        out_hbm.at[c].at[pl.ds(s * ROWS_PER_TILE, ROWS_PER_TILE)],
    )


RB = 1000


def _dis_block(dp_ref):
    d = dp_ref[0, :, 0:1] + dp_ref[1, :, 0:1] + 1.0
    return lax.rsqrt(d)


def _tc1_body(x_ref, w_ref, dp_ref, y_ref):
    dis = _dis_block(dp_ref)
    y_ref[...] = jnp.dot(x_ref[...], w_ref[...],
                         preferred_element_type=jnp.float32) * dis


def _tc2_body(a_ref, y_ref, dp_ref, b_ref, w_ref, o_ref):
    dis = _dis_block(dp_ref)
    h = jnp.maximum(dis * (a_ref[0] + a_ref[1] + y_ref[...]) + b_ref[...], 0.0)
    o_ref[...] = jnp.dot(h, w_ref[...], preferred_element_type=jnp.float32) * dis


def _tc3_body(a_ref, y_ref, dp_ref, b_ref, w_ref, bo_ref, o_ref):
    dis = _dis_block(dp_ref)
    h = jnp.maximum(dis * (a_ref[0] + a_ref[1] + y_ref[...]) + b_ref[...], 0.0)
    o_ref[...] = jnp.dot(h, w_ref[...],
                         preferred_element_type=jnp.float32) + bo_ref[...]


_dp_spec = pl.BlockSpec((NC, RB, DEG_W), lambda i: (0, i, 0))
_acc_spec = pl.BlockSpec((NC, RB, D), lambda i: (0, i, 0))
_rowblk = pl.BlockSpec((RB, D), lambda i: (i, 0))

_tc1 = pl.pallas_call(
    _tc1_body,
    grid=(N // RB,),
    in_specs=[
        _rowblk,
        pl.BlockSpec((D, D), lambda i: (0, 0)),
        _dp_spec,
    ],
    out_specs=_rowblk,
    out_shape=jax.ShapeDtypeStruct((N, D), jnp.float32),
)

_tc2 = pl.pallas_call(
    _tc2_body,
    grid=(N // RB,),
    in_specs=[
        _acc_spec,
        _rowblk,
        _dp_spec,
        pl.BlockSpec((1, D), lambda i: (0, 0)),
        pl.BlockSpec((D, D), lambda i: (0, 0)),
    ],
    out_specs=_rowblk,
    out_shape=jax.ShapeDtypeStruct((N, D), jnp.float32),
)

_tc3 = pl.pallas_call(
    _tc3_body,
    grid=(N // RB,),
    in_specs=[
        _acc_spec,
        _rowblk,
        _dp_spec,
        pl.BlockSpec((1, D), lambda i: (0, 0)),
        pl.BlockSpec((D, C), lambda i: (0, 0)),
        pl.BlockSpec((1, C), lambda i: (0, 0)),
    ],
    out_specs=pl.BlockSpec((RB, C), lambda i: (i, 0)),
    out_shape=jax.ShapeDtypeStruct((N, C), jnp.float32),
)


def kernel(x, edge_index, W1, b1, W2, b2, W_out, b_out):
    row = edge_index[0].astype(jnp.int32)
    col = edge_index[1].astype(jnp.int32)
    n_tiles = NC * NS
    pad_per_tile = E_PAD // n_tiles - E // n_tiles
    pad_cols = jnp.broadcast_to(N + jnp.arange(pad_per_tile, dtype=jnp.int32),
                                (n_tiles, pad_per_tile))
    pad_rows = (jnp.arange(n_tiles, dtype=jnp.int32)[:, None] * pad_per_tile
                + jnp.arange(pad_per_tile, dtype=jnp.int32)[None, :]) % N
    row_p = jnp.concatenate(
        [row.reshape(n_tiles, E // n_tiles), pad_rows],
        axis=1).reshape(NC, NS, NCHUNK, CHUNK)
    col_p = jnp.concatenate(
        [col.reshape(n_tiles, E // n_tiles), pad_cols],
        axis=1).reshape(NC, NS, NCHUNK, CHUNK)

    onesW = jnp.ones((CHUNK, DEG_W), jnp.float32)
    zerosD = jnp.zeros((ROWS_PER_TILE, D), jnp.float32)

    deg_parts = _sc_degree(col_p, onesW, zerosD)
    y1 = _tc1(x, W1, deg_parts)
    acc1 = _sc_propagate(y1, row_p, col_p, zerosD)
    y2 = _tc2(acc1, y1, deg_parts, b1.reshape(1, D), W2)
    acc2 = _sc_propagate(y2, row_p, col_p, zerosD)
    out = _tc3(acc2, y2, deg_parts, b2.reshape(1, D), W_out.T, b_out.reshape(1, C))
    return out


# speedup vs baseline: 26.5001x; 1.0018x over previous
import functools

import jax
import jax.numpy as jnp
from jax import lax
from jax.experimental import pallas as pl
from jax.experimental.pallas import tpu as pltpu
from jax.experimental.pallas import tpu_sc as plsc

N = 10000
E = 320000
D = 128
C = 64
NC = 2
NS = 16
CHUNK = 128
NCHUNK = 80
E_PAD = NC * NS * NCHUNK * CHUNK
N_PAD = 10240
ROWS_PER_TILE = N_PAD // NS
DEG_W = 128

_sc_mesh = plsc.VectorSubcoreMesh(core_axis_name="c", subcore_axis_name="s")


@functools.partial(
    pl.kernel,
    mesh=_sc_mesh,
    out_type=jax.ShapeDtypeStruct((NC, N_PAD, DEG_W), jnp.float32),
    scratch_types=[
        pltpu.VMEM((NCHUNK, CHUNK), jnp.int32),
        pltpu.VMEM((CHUNK, DEG_W), jnp.float32),
        pltpu.VMEM_SHARED((N_PAD, DEG_W), jnp.float32),
    ],
)
def _sc_degree(col_hbm, ones_hbm, zeros_hbm, out_hbm, col_v, ones_v, acc_sh):
    c = lax.axis_index("c")
    s = lax.axis_index("s")
    pltpu.sync_copy(col_hbm.at[c, s], col_v)
    pltpu.sync_copy(ones_hbm, ones_v)
    pltpu.sync_copy(zeros_hbm, acc_sh.at[pl.ds(s * ROWS_PER_TILE, ROWS_PER_TILE)])
    plsc.subcore_barrier()

    def body(j, carry):
        pltpu.sync_copy(ones_v, acc_sh.at[col_v.at[j]], add=True)
        return carry

    lax.fori_loop(0, NCHUNK, body, 0)
    plsc.subcore_barrier()
    pltpu.sync_copy(
        acc_sh.at[pl.ds(s * ROWS_PER_TILE, ROWS_PER_TILE)],
        out_hbm.at[c].at[pl.ds(s * ROWS_PER_TILE, ROWS_PER_TILE)],
    )


NPHASE = 2
PCHUNK = NCHUNK // NPHASE


@functools.partial(
    pl.kernel,
    mesh=_sc_mesh,
    out_type=jax.ShapeDtypeStruct((NC, N_PAD, D), jnp.float32),
    scratch_types=[
        pltpu.VMEM((PCHUNK, CHUNK), jnp.int32),
        pltpu.VMEM((PCHUNK, CHUNK), jnp.int32),
        pltpu.VMEM((CHUNK, D), jnp.float32),
        pltpu.VMEM((CHUNK, D), jnp.float32),
        pltpu.VMEM_SHARED((N_PAD, D), jnp.float32),
        pltpu.SemaphoreType.DMA,
        pltpu.SemaphoreType.DMA,
    ],
)
def _sc_propagate(y_hbm, row_hbm, col_hbm, zeros_hbm, out_hbm,
                  row_v, col_v, msg0_v, msg1_v, acc_sh, sem0, sem1):
    c = lax.axis_index("c")
    s = lax.axis_index("s")
    pltpu.sync_copy(zeros_hbm, acc_sh.at[pl.ds(s * ROWS_PER_TILE, ROWS_PER_TILE)])
    plsc.subcore_barrier()

    for p in range(NPHASE):
        pltpu.sync_copy(row_hbm.at[c, s].at[pl.ds(p * PCHUNK, PCHUNK)], row_v)
        pltpu.sync_copy(col_hbm.at[c, s].at[pl.ds(p * PCHUNK, PCHUNK)], col_v)
        pltpu.async_copy(y_hbm.at[row_v.at[0]], msg0_v, sem0)

        def body(g, carry):
            j0 = 2 * g
            pltpu.async_copy(y_hbm.at[row_v.at[j0 + 1]], msg1_v, sem1)
            pltpu.make_async_copy(y_hbm.at[row_v.at[j0]], msg0_v, sem0).wait()
            pltpu.sync_copy(msg0_v, acc_sh.at[col_v.at[j0]], add=True)

            @pl.when(g < PCHUNK // 2 - 1)
            def _():
                pltpu.async_copy(y_hbm.at[row_v.at[j0 + 2]], msg0_v, sem0)

            pltpu.make_async_copy(y_hbm.at[row_v.at[j0 + 1]], msg1_v, sem1).wait()
            pltpu.sync_copy(msg1_v, acc_sh.at[col_v.at[j0 + 1]], add=True)
            return carry

        lax.fori_loop(0, PCHUNK // 2, body, 0)
    plsc.subcore_barrier()
    pltpu.sync_copy(
        acc_sh.at[pl.ds(s * ROWS_PER_TILE, ROWS_PER_TILE)],
        out_hbm.at[c].at[pl.ds(s * ROWS_PER_TILE, ROWS_PER_TILE)],
    )


RB = 1000


def _dis_block(dp_ref):
    d = dp_ref[0, :, 0:1] + dp_ref[1, :, 0:1] + 1.0
    return lax.rsqrt(d)


def _tc1_body(x_ref, w_ref, dp_ref, y_ref):
    dis = _dis_block(dp_ref)
    y_ref[...] = jnp.dot(x_ref[...], w_ref[...],
                         preferred_element_type=jnp.float32) * dis


def _tc2_body(a_ref, y_ref, dp_ref, b_ref, w_ref, o_ref):
    dis = _dis_block(dp_ref)
    h = jnp.maximum(dis * (a_ref[0] + a_ref[1] + y_ref[...]) + b_ref[...], 0.0)
    o_ref[...] = jnp.dot(h, w_ref[...], preferred_element_type=jnp.float32) * dis


def _tc3_body(a_ref, y_ref, dp_ref, b_ref, w_ref, bo_ref, o_ref):
    dis = _dis_block(dp_ref)
    h = jnp.maximum(dis * (a_ref[0] + a_ref[1] + y_ref[...]) + b_ref[...], 0.0)
    o_ref[...] = jnp.dot(h, w_ref[...],
                         preferred_element_type=jnp.float32) + bo_ref[...]


DPW = 16
_dp_spec = pl.BlockSpec((NC, RB, DPW), lambda i: (0, i, 0))
_acc_spec = pl.BlockSpec((NC, RB, D), lambda i: (0, i, 0))
_rowblk = pl.BlockSpec((RB, D), lambda i: (i, 0))

_tc1 = pl.pallas_call(
    _tc1_body,
    grid=(N // RB,),
    in_specs=[
        _rowblk,
        pl.BlockSpec((D, D), lambda i: (0, 0)),
        _dp_spec,
    ],
    out_specs=_rowblk,
    out_shape=jax.ShapeDtypeStruct((N, D), jnp.float32),
)

_tc2 = pl.pallas_call(
    _tc2_body,
    grid=(N // RB,),
    in_specs=[
        _acc_spec,
        _rowblk,
        _dp_spec,
        pl.BlockSpec((1, D), lambda i: (0, 0)),
        pl.BlockSpec((D, D), lambda i: (0, 0)),
    ],
    out_specs=_rowblk,
    out_shape=jax.ShapeDtypeStruct((N, D), jnp.float32),
)

_tc3 = pl.pallas_call(
    _tc3_body,
    grid=(N // RB,),
    in_specs=[
        _acc_spec,
        _rowblk,
        _dp_spec,
        pl.BlockSpec((1, D), lambda i: (0, 0)),
        pl.BlockSpec((D, C), lambda i: (0, 0)),
        pl.BlockSpec((1, C), lambda i: (0, 0)),
    ],
    out_specs=pl.BlockSpec((RB, C), lambda i: (i, 0)),
    out_shape=jax.ShapeDtypeStruct((N, C), jnp.float32),
)


def kernel(x, edge_index, W1, b1, W2, b2, W_out, b_out):
    row = edge_index[0].astype(jnp.int32)
    col = edge_index[1].astype(jnp.int32)
    n_tiles = NC * NS
    pad_per_tile = E_PAD // n_tiles - E // n_tiles
    pad_cols = jnp.broadcast_to(N + jnp.arange(pad_per_tile, dtype=jnp.int32),
                                (n_tiles, pad_per_tile))
    pad_rows = (jnp.arange(n_tiles, dtype=jnp.int32)[:, None] * pad_per_tile
                + jnp.arange(pad_per_tile, dtype=jnp.int32)[None, :]) % N
    row_p = jnp.concatenate(
        [row.reshape(n_tiles, E // n_tiles), pad_rows],
        axis=1).reshape(NC, NS, NCHUNK, CHUNK)
    col_p = jnp.concatenate(
        [col.reshape(n_tiles, E // n_tiles), pad_cols],
        axis=1).reshape(NC, NS, NCHUNK, CHUNK)

    onesW = jnp.ones((CHUNK, DEG_W), jnp.float32)
    zerosD = jnp.zeros((ROWS_PER_TILE, D), jnp.float32)

    deg_parts = _sc_degree(col_p, onesW, zerosD)[:, :, :DPW]
    y1 = _tc1(x, W1, deg_parts)
    acc1 = _sc_propagate(y1, row_p, col_p, zerosD)
    y2 = _tc2(acc1, y1, deg_parts, b1.reshape(1, D), W2)
    acc2 = _sc_propagate(y2, row_p, col_p, zerosD)
    out = _tc3(acc2, y2, deg_parts, b2.reshape(1, D), W_out.T, b_out.reshape(1, C))
    return out


# speedup vs baseline: 26.8576x; 1.0135x over previous
import functools

import jax
import jax.numpy as jnp
from jax import lax
from jax.experimental import pallas as pl
from jax.experimental.pallas import tpu as pltpu
from jax.experimental.pallas import tpu_sc as plsc

N = 10000
E = 320000
D = 128
C = 64
NC = 2
NS = 16
CHUNK = 128
NCHUNK = 80
E_PAD = NC * NS * NCHUNK * CHUNK
N_PAD = 10240
ROWS_PER_TILE = N_PAD // NS
DEG_W = 128

_sc_mesh = plsc.VectorSubcoreMesh(core_axis_name="c", subcore_axis_name="s")


@functools.partial(
    pl.kernel,
    mesh=_sc_mesh,
    out_type=jax.ShapeDtypeStruct((NC, N_PAD, DEG_W), jnp.float32),
    scratch_types=[
        pltpu.VMEM((NCHUNK, CHUNK), jnp.int32),
        pltpu.VMEM((CHUNK, DEG_W), jnp.float32),
        pltpu.VMEM_SHARED((N_PAD, DEG_W), jnp.float32),
        pltpu.SemaphoreType.DMA,
    ],
)
def _sc_degree(col_hbm, ones_hbm, zeros_hbm, out_hbm, col_v, ones_v, acc_sh,
               zsem):
    c = lax.axis_index("c")
    s = lax.axis_index("s")
    zcp = pltpu.async_copy(
        zeros_hbm, acc_sh.at[pl.ds(s * ROWS_PER_TILE, ROWS_PER_TILE)], zsem)
    pltpu.sync_copy(col_hbm.at[c, s], col_v)
    pltpu.sync_copy(ones_hbm, ones_v)
    zcp.wait()
    plsc.subcore_barrier()

    def body(j, carry):
        pltpu.sync_copy(ones_v, acc_sh.at[col_v.at[j]], add=True)
        return carry

    lax.fori_loop(0, NCHUNK, body, 0)
    plsc.subcore_barrier()
    pltpu.sync_copy(
        acc_sh.at[pl.ds(s * ROWS_PER_TILE, ROWS_PER_TILE)],
        out_hbm.at[c].at[pl.ds(s * ROWS_PER_TILE, ROWS_PER_TILE)],
    )


NPHASE = 2
PCHUNK = NCHUNK // NPHASE


@functools.partial(
    pl.kernel,
    mesh=_sc_mesh,
    out_type=jax.ShapeDtypeStruct((NC, N_PAD, D), jnp.float32),
    scratch_types=[
        pltpu.VMEM((PCHUNK, CHUNK), jnp.int32),
        pltpu.VMEM((PCHUNK, CHUNK), jnp.int32),
        pltpu.VMEM((CHUNK, D), jnp.float32),
        pltpu.VMEM((CHUNK, D), jnp.float32),
        pltpu.VMEM_SHARED((N_PAD, D), jnp.float32),
        pltpu.SemaphoreType.DMA,
        pltpu.SemaphoreType.DMA,
        pltpu.SemaphoreType.DMA,
    ],
)
def _sc_propagate(y_hbm, row_hbm, col_hbm, zeros_hbm, out_hbm,
                  row_v, col_v, msg0_v, msg1_v, acc_sh, sem0, sem1, zsem):
    c = lax.axis_index("c")
    s = lax.axis_index("s")
    zcp = pltpu.async_copy(
        zeros_hbm, acc_sh.at[pl.ds(s * ROWS_PER_TILE, ROWS_PER_TILE)], zsem)

    for p in range(NPHASE):
        pltpu.sync_copy(row_hbm.at[c, s].at[pl.ds(p * PCHUNK, PCHUNK)], row_v)
        pltpu.sync_copy(col_hbm.at[c, s].at[pl.ds(p * PCHUNK, PCHUNK)], col_v)
        if p == 0:
            zcp.wait()
            plsc.subcore_barrier()
        pltpu.async_copy(y_hbm.at[row_v.at[0]], msg0_v, sem0)

        def body(g, carry):
            j0 = 2 * g
            pltpu.async_copy(y_hbm.at[row_v.at[j0 + 1]], msg1_v, sem1)
            pltpu.make_async_copy(y_hbm.at[row_v.at[j0]], msg0_v, sem0).wait()
            pltpu.sync_copy(msg0_v, acc_sh.at[col_v.at[j0]], add=True)

            @pl.when(g < PCHUNK // 2 - 1)
            def _():
                pltpu.async_copy(y_hbm.at[row_v.at[j0 + 2]], msg0_v, sem0)

            pltpu.make_async_copy(y_hbm.at[row_v.at[j0 + 1]], msg1_v, sem1).wait()
            pltpu.sync_copy(msg1_v, acc_sh.at[col_v.at[j0 + 1]], add=True)
            return carry

        lax.fori_loop(0, PCHUNK // 2, body, 0)
    plsc.subcore_barrier()
    pltpu.sync_copy(
        acc_sh.at[pl.ds(s * ROWS_PER_TILE, ROWS_PER_TILE)],
        out_hbm.at[c].at[pl.ds(s * ROWS_PER_TILE, ROWS_PER_TILE)],
    )


RB = 1000


def _dis_block(dp_ref):
    d = dp_ref[0, :, 0:1] + dp_ref[1, :, 0:1] + 1.0
    return lax.rsqrt(d)


def _tcmm_body(x_ref, w_ref, y_ref):
    y_ref[...] = jnp.dot(x_ref[...], w_ref[...],
                         preferred_element_type=jnp.float32)


def _tcscale_body(u_ref, dp_ref, y_ref):
    y_ref[...] = u_ref[...] * _dis_block(dp_ref)


def _tc2_body(a_ref, y_ref, dp_ref, b_ref, w_ref, o_ref):
    dis = _dis_block(dp_ref)
    h = jnp.maximum(dis * (a_ref[0] + a_ref[1] + y_ref[...]) + b_ref[...], 0.0)
    o_ref[...] = jnp.dot(h, w_ref[...], preferred_element_type=jnp.float32) * dis


def _tc3_body(a_ref, y_ref, dp_ref, b_ref, w_ref, bo_ref, o_ref):
    dis = _dis_block(dp_ref)
    h = jnp.maximum(dis * (a_ref[0] + a_ref[1] + y_ref[...]) + b_ref[...], 0.0)
    o_ref[...] = jnp.dot(h, w_ref[...],
                         preferred_element_type=jnp.float32) + bo_ref[...]


DPW = 16
_dp_spec = pl.BlockSpec((NC, RB, DPW), lambda i: (0, i, 0))
_acc_spec = pl.BlockSpec((NC, RB, D), lambda i: (0, i, 0))
_rowblk = pl.BlockSpec((RB, D), lambda i: (i, 0))

_tc_mm = pl.pallas_call(
    _tcmm_body,
    grid=(N // RB,),
    in_specs=[
        _rowblk,
        pl.BlockSpec((D, D), lambda i: (0, 0)),
    ],
    out_specs=_rowblk,
    out_shape=jax.ShapeDtypeStruct((N, D), jnp.float32),
)

_tc_scale = pl.pallas_call(
    _tcscale_body,
    grid=(N // RB,),
    in_specs=[
        _rowblk,
        _dp_spec,
    ],
    out_specs=_rowblk,
    out_shape=jax.ShapeDtypeStruct((N, D), jnp.float32),
)

_tc2 = pl.pallas_call(
    _tc2_body,
    grid=(N // RB,),
    in_specs=[
        _acc_spec,
        _rowblk,
        _dp_spec,
        pl.BlockSpec((1, D), lambda i: (0, 0)),
        pl.BlockSpec((D, D), lambda i: (0, 0)),
    ],
    out_specs=_rowblk,
    out_shape=jax.ShapeDtypeStruct((N, D), jnp.float32),
)

_tc3 = pl.pallas_call(
    _tc3_body,
    grid=(N // RB,),
    in_specs=[
        _acc_spec,
        _rowblk,
        _dp_spec,
        pl.BlockSpec((1, D), lambda i: (0, 0)),
        pl.BlockSpec((D, C), lambda i: (0, 0)),
        pl.BlockSpec((1, C), lambda i: (0, 0)),
    ],
    out_specs=pl.BlockSpec((RB, C), lambda i: (i, 0)),
    out_shape=jax.ShapeDtypeStruct((N, C), jnp.float32),
)


def kernel(x, edge_index, W1, b1, W2, b2, W_out, b_out):
    row = edge_index[0].astype(jnp.int32)
    col = edge_index[1].astype(jnp.int32)
    n_tiles = NC * NS
    pad_per_tile = E_PAD // n_tiles - E // n_tiles
    pad_cols = jnp.broadcast_to(N + jnp.arange(pad_per_tile, dtype=jnp.int32),
                                (n_tiles, pad_per_tile))
    pad_rows = (jnp.arange(n_tiles, dtype=jnp.int32)[:, None] * pad_per_tile
                + jnp.arange(pad_per_tile, dtype=jnp.int32)[None, :]) % N
    row_p = jnp.concatenate(
        [row.reshape(n_tiles, E // n_tiles), pad_rows],
        axis=1).reshape(NC, NS, NCHUNK, CHUNK)
    col_p = jnp.concatenate(
        [col.reshape(n_tiles, E // n_tiles), pad_cols],
        axis=1).reshape(NC, NS, NCHUNK, CHUNK)

    onesW = jnp.ones((CHUNK, DEG_W), jnp.float32)
    zerosD = jnp.zeros((ROWS_PER_TILE, D), jnp.float32)

    u1 = _tc_mm(x, W1)
    deg_parts = _sc_degree(col_p, onesW, zerosD)[:, :, :DPW]
    y1 = _tc_scale(u1, deg_parts)
    acc1 = _sc_propagate(y1, row_p, col_p, zerosD)
    y2 = _tc2(acc1, y1, deg_parts, b1.reshape(1, D), W2)
    acc2 = _sc_propagate(y2, row_p, col_p, zerosD)
    out = _tc3(acc2, y2, deg_parts, b2.reshape(1, D), W_out.T, b_out.reshape(1, C))
    return out


# speedup vs baseline: 27.0806x; 1.0083x over previous
import functools

import jax
import jax.numpy as jnp
from jax import lax
from jax.experimental import pallas as pl
from jax.experimental.pallas import tpu as pltpu
from jax.experimental.pallas import tpu_sc as plsc

N = 10000
E = 320000
D = 128
C = 64
NC = 2
NS = 16
CHUNK = 128
NCHUNK = 80
E_PAD = NC * NS * NCHUNK * CHUNK
N_PAD = 10240
ROWS_PER_TILE = N_PAD // NS
DEG_W = 128

_sc_mesh = plsc.VectorSubcoreMesh(core_axis_name="c", subcore_axis_name="s")


@functools.partial(
    pl.kernel,
    mesh=_sc_mesh,
    out_type=jax.ShapeDtypeStruct((NC, N_PAD, DEG_W), jnp.float32),
    scratch_types=[
        pltpu.VMEM((NCHUNK, CHUNK), jnp.int32),
        pltpu.VMEM((CHUNK, DEG_W), jnp.float32),
        pltpu.VMEM_SHARED((N_PAD, DEG_W), jnp.float32),
        pltpu.SemaphoreType.DMA,
    ],
)
def _sc_degree(col_hbm, ones_hbm, zeros_hbm, out_hbm, col_v, ones_v, acc_sh,
               zsem):
    c = lax.axis_index("c")
    s = lax.axis_index("s")
    zcp = pltpu.async_copy(
        zeros_hbm, acc_sh.at[pl.ds(s * ROWS_PER_TILE, ROWS_PER_TILE)], zsem)
    pltpu.sync_copy(col_hbm.at[c, s], col_v)
    pltpu.sync_copy(ones_hbm, ones_v)
    zcp.wait()
    plsc.subcore_barrier()

    def body(j, carry):
        pltpu.sync_copy(ones_v, acc_sh.at[col_v.at[j]], add=True)
        return carry

    lax.fori_loop(0, NCHUNK, body, 0)
    plsc.subcore_barrier()
    pltpu.sync_copy(
        acc_sh.at[pl.ds(s * ROWS_PER_TILE, ROWS_PER_TILE)],
        out_hbm.at[c].at[pl.ds(s * ROWS_PER_TILE, ROWS_PER_TILE)],
    )


NPHASE = 2
PCHUNK = NCHUNK // NPHASE
E_PHASE = E // (NC * NS) // NPHASE
PAD_PHASE = PCHUNK * CHUNK - E_PHASE


@functools.partial(
    pl.kernel,
    mesh=_sc_mesh,
    out_type=jax.ShapeDtypeStruct((NC, N_PAD, D), jnp.float32),
    scratch_types=[
        pltpu.VMEM((PCHUNK, CHUNK), jnp.int32),
        pltpu.VMEM((PCHUNK, CHUNK), jnp.int32),
        pltpu.VMEM((CHUNK, D), jnp.float32),
        pltpu.VMEM((CHUNK, D), jnp.float32),
        pltpu.VMEM_SHARED((N_PAD, D), jnp.float32),
        pltpu.SemaphoreType.DMA,
        pltpu.SemaphoreType.DMA,
        pltpu.SemaphoreType.DMA,
    ],
)
def _sc_propagate(y_hbm, row_hbm, col_hbm, zeros_hbm, out_hbm,
                  row_v, col_v, msg0_v, msg1_v, acc_sh, sem0, sem1, zsem):
    c = lax.axis_index("c")
    s = lax.axis_index("s")
    zcp = pltpu.async_copy(
        zeros_hbm, acc_sh.at[pl.ds(s * ROWS_PER_TILE, ROWS_PER_TILE)], zsem)

    for p in range(NPHASE):
        pltpu.sync_copy(row_hbm.at[c, s].at[pl.ds(p * PCHUNK, PCHUNK)], row_v)
        pltpu.async_copy(y_hbm.at[row_v.at[0]], msg0_v, sem0)
        pltpu.sync_copy(col_hbm.at[c, s].at[pl.ds(p * PCHUNK, PCHUNK)], col_v)
        if p == 0:
            zcp.wait()
            plsc.subcore_barrier()

        def body(g, carry):
            j0 = 2 * g
            pltpu.async_copy(y_hbm.at[row_v.at[j0 + 1]], msg1_v, sem1)
            pltpu.make_async_copy(y_hbm.at[row_v.at[j0]], msg0_v, sem0).wait()
            pltpu.sync_copy(msg0_v, acc_sh.at[col_v.at[j0]], add=True)

            @pl.when(g < PCHUNK // 2 - 1)
            def _():
                pltpu.async_copy(y_hbm.at[row_v.at[j0 + 2]], msg0_v, sem0)

            pltpu.make_async_copy(y_hbm.at[row_v.at[j0 + 1]], msg1_v, sem1).wait()
            pltpu.sync_copy(msg1_v, acc_sh.at[col_v.at[j0 + 1]], add=True)
            return carry

        lax.fori_loop(0, PCHUNK // 2, body, 0)
    plsc.subcore_barrier()
    pltpu.sync_copy(
        acc_sh.at[pl.ds(s * ROWS_PER_TILE, ROWS_PER_TILE)],
        out_hbm.at[c].at[pl.ds(s * ROWS_PER_TILE, ROWS_PER_TILE)],
    )


RB = 1000


def _dis_block(dp_ref):
    d = dp_ref[0, :, 0:1] + dp_ref[1, :, 0:1] + 1.0
    return lax.rsqrt(d)


def _tcmm_body(x_ref, w_ref, y_ref):
    y_ref[...] = jnp.dot(x_ref[...], w_ref[...],
                         preferred_element_type=jnp.float32)


def _tcscale_body(u_ref, dp_ref, y_ref):
    y_ref[...] = u_ref[...] * _dis_block(dp_ref)


def _tc2_body(a_ref, y_ref, dp_ref, b_ref, w_ref, o_ref):
    dis = _dis_block(dp_ref)
    h = jnp.maximum(dis * (a_ref[0] + a_ref[1] + y_ref[...]) + b_ref[...], 0.0)
    o_ref[...] = jnp.dot(h, w_ref[...], preferred_element_type=jnp.float32) * dis


def _tc3_body(a_ref, y_ref, dp_ref, b_ref, w_ref, bo_ref, o_ref):
    dis = _dis_block(dp_ref)
    h = jnp.maximum(dis * (a_ref[0] + a_ref[1] + y_ref[...]) + b_ref[...], 0.0)
    o_ref[...] = jnp.dot(h, w_ref[...],
                         preferred_element_type=jnp.float32) + bo_ref[...]


DPW = 16
_dp_spec = pl.BlockSpec((NC, RB, DPW), lambda i: (0, i, 0))
_acc_spec = pl.BlockSpec((NC, RB, D), lambda i: (0, i, 0))
_rowblk = pl.BlockSpec((RB, D), lambda i: (i, 0))

_tc_mm = pl.pallas_call(
    _tcmm_body,
    grid=(N // RB,),
    in_specs=[
        _rowblk,
        pl.BlockSpec((D, D), lambda i: (0, 0)),
    ],
    out_specs=_rowblk,
    out_shape=jax.ShapeDtypeStruct((N, D), jnp.float32),
)

_tc_scale = pl.pallas_call(
    _tcscale_body,
    grid=(N // RB,),
    in_specs=[
        _rowblk,
        _dp_spec,
    ],
    out_specs=_rowblk,
    out_shape=jax.ShapeDtypeStruct((N, D), jnp.float32),
)

_tc2 = pl.pallas_call(
    _tc2_body,
    grid=(N // RB,),
    in_specs=[
        _acc_spec,
        _rowblk,
        _dp_spec,
        pl.BlockSpec((1, D), lambda i: (0, 0)),
        pl.BlockSpec((D, D), lambda i: (0, 0)),
    ],
    out_specs=_rowblk,
    out_shape=jax.ShapeDtypeStruct((N, D), jnp.float32),
)

_tc3 = pl.pallas_call(
    _tc3_body,
    grid=(N // RB,),
    in_specs=[
        _acc_spec,
        _rowblk,
        _dp_spec,
        pl.BlockSpec((1, D), lambda i: (0, 0)),
        pl.BlockSpec((D, C), lambda i: (0, 0)),
        pl.BlockSpec((1, C), lambda i: (0, 0)),
    ],
    out_specs=pl.BlockSpec((RB, C), lambda i: (i, 0)),
    out_shape=jax.ShapeDtypeStruct((N, C), jnp.float32),
)


def kernel(x, edge_index, W1, b1, W2, b2, W_out, b_out):
    row = edge_index[0].astype(jnp.int32)
    col = edge_index[1].astype(jnp.int32)
    n_tiles = NC * NS
    pad_rows = ((jnp.arange(n_tiles * NPHASE, dtype=jnp.int32)[:, None]
                 * PAD_PHASE
                 + jnp.arange(PAD_PHASE, dtype=jnp.int32)[None, :]) % N
                ).reshape(n_tiles, NPHASE, PAD_PHASE)
    row_p = jnp.concatenate(
        [row.reshape(n_tiles, NPHASE, E_PHASE), pad_rows],
        axis=2).reshape(NC, NS, NCHUNK, CHUNK)
    col_p = jnp.concatenate(
        [col.reshape(n_tiles, NPHASE, E_PHASE),
         jnp.broadcast_to(
             N + jnp.arange(PAD_PHASE, dtype=jnp.int32),
             (n_tiles, NPHASE, PAD_PHASE))],
        axis=2).reshape(NC, NS, NCHUNK, CHUNK)

    onesW = jnp.ones((CHUNK, DEG_W), jnp.float32)
    zerosD = jnp.zeros((ROWS_PER_TILE, D), jnp.float32)

    u1 = _tc_mm(x, W1)
    deg_parts = _sc_degree(col_p, onesW, zerosD)[:, :, :DPW]
    y1 = _tc_scale(u1, deg_parts)
    acc1 = _sc_propagate(y1, row_p, col_p, zerosD)
    y2 = _tc2(acc1, y1, deg_parts, b1.reshape(1, D), W2)
    acc2 = _sc_propagate(y2, row_p, col_p, zerosD)
    out = _tc3(acc2, y2, deg_parts, b2.reshape(1, D), W_out.T, b_out.reshape(1, C))
    return out
